# baseline (speedup 1.0000x reference)
import math as _math

import jax
import jax.numpy as jnp
from jax import lax
from jax.experimental import pallas as pl
from jax.experimental.pallas import tpu as pltpu
from jax.experimental.pallas import tpu_sc as plsc

_SIZE = 16384
_N = 4096
_SMOOTH = 0.1
_CONF = 1.0 - _SMOOTH
_S = _SMOOTH / (_SIZE - 2)
_C0 = (_SIZE - 2) * _S * _math.log(_S) + _CONF * _math.log(_CONF)

_R = 256

_info = plsc.get_sparse_core_info()
_NC, _NS, _L = _info.num_cores, _info.num_subcores, _info.num_lanes
_NW = _NC * _NS
_BPW = _N // _NW


def _sc_count_body(tgt_hbm, out_hbm, tgt_v, part_v):
    wid = lax.axis_index("s") * _NC + lax.axis_index("c")
    base = wid * _BPW
    pltpu.sync_copy(tgt_hbm.at[pl.ds(base, _BPW)], tgt_v)
    acc = jnp.zeros((_L,), jnp.float32)
    for c in range(_BPW // _L):
        t = tgt_v[pl.ds(c * _L, _L)]
        acc = acc + jnp.where(t != 0, jnp.float32(_C0), jnp.float32(0.0))
    part_v[...] = acc
    pltpu.sync_copy(part_v, out_hbm.at[wid])


_sc_count = pl.kernel(
    _sc_count_body,
    out_type=jax.ShapeDtypeStruct((_NW, _L), jnp.float32),
    mesh=plsc.VectorSubcoreMesh(core_axis_name="c", subcore_axis_name="s"),
    scratch_types=[
        pltpu.VMEM((_BPW,), jnp.int32),
        pltpu.VMEM((_L,), jnp.float32),
    ],
)


def _tc_body(tgt_ref, parts_ref, x_ref, out_ref):
    i = pl.program_id(0)
    t = tgt_ref[0, 0, :]
    x = x_ref[...]
    cols = lax.broadcasted_iota(jnp.int32, (_R, _SIZE), 1)
    td = jnp.where(cols == t[:, None], jnp.float32(_CONF), jnp.float32(_S))
    td = jnp.where(cols == 0, jnp.float32(0.0), td)
    rowdot = jnp.sum(td * x, axis=1)
    pad_fix = jnp.where(t == 0, rowdot, jnp.float32(0.0))
    val = jnp.sum(pad_fix) - jnp.sum(rowdot)

    @pl.when(i == 0)
    def _():
        out_ref[0, 0] = jnp.sum(parts_ref[...])

    out_ref[0, 0] += val


@jax.jit
def kernel(x, target):
    nr = _N // _R
    tgt = target.astype(jnp.int32)
    parts = _sc_count(tgt)
    dense = pl.pallas_call(
        _tc_body,
        grid=(nr,),
        in_specs=[
            pl.BlockSpec((1, 1, _R), lambda i: (i, 0, 0)),
            pl.BlockSpec((_NW, _L), lambda i: (0, 0)),
            pl.BlockSpec((_R, _SIZE), lambda i: (i, 0)),
        ],
        out_specs=pl.BlockSpec(
            (1, 1), lambda i: (0, 0), memory_space=pltpu.SMEM),
        out_shape=jax.ShapeDtypeStruct((1, 1), jnp.float32),
    )(tgt.reshape(nr, 1, _R), parts, x)
    return dense[0, 0]


# speedup vs baseline: 1.0370x; 1.0370x over previous
import math as _math

import jax
import jax.numpy as jnp
from jax import lax
from jax.experimental import pallas as pl
from jax.experimental.pallas import tpu as pltpu
from jax.experimental.pallas import tpu_sc as plsc

_SIZE = 16384
_N = 4096
_SMOOTH = 0.1
_CONF = 1.0 - _SMOOTH
_S = _SMOOTH / (_SIZE - 2)
_C0 = (_SIZE - 2) * _S * _math.log(_S) + _CONF * _math.log(_CONF)

_R = 256

_info = plsc.get_sparse_core_info()
_NC, _NS, _L = _info.num_cores, _info.num_subcores, _info.num_lanes
_NW = _NC * _NS
_BPW = _N // _NW


def _tc_body(tgt_ref, x_ref, u_ref):
    t = tgt_ref[0, 0, :]
    x = x_ref[...]
    cols = lax.broadcasted_iota(jnp.int32, (_R, _SIZE), 1)
    td = jnp.where(cols == t[:, None], jnp.float32(_CONF), jnp.float32(_S))
    td = jnp.where(cols == 0, jnp.float32(0.0), td)
    u_ref[...] = jnp.sum(td * x, axis=1)


def _sc_finalize_body(tgt_hbm, u_hbm, out_hbm, tgt_v, u_v, part_v):
    wid = lax.axis_index("s") * _NC + lax.axis_index("c")
    base = wid * _BPW
    pltpu.sync_copy(tgt_hbm.at[pl.ds(base, _BPW)], tgt_v)
    pltpu.sync_copy(u_hbm.at[pl.ds(base, _BPW)], u_v)
    acc = jnp.zeros((_L,), jnp.float32)
    for c in range(_BPW // _L):
        t = tgt_v[pl.ds(c * _L, _L)]
        u = u_v[pl.ds(c * _L, _L)]
        acc = acc + jnp.where(t != 0, jnp.float32(_C0) - u, jnp.float32(0.0))
    part_v[...] = acc
    pltpu.sync_copy(part_v, out_hbm.at[wid])


_sc_finalize = pl.kernel(
    _sc_finalize_body,
    out_type=jax.ShapeDtypeStruct((_NW, _L), jnp.float32),
    mesh=plsc.VectorSubcoreMesh(core_axis_name="c", subcore_axis_name="s"),
    scratch_types=[
        pltpu.VMEM((_BPW,), jnp.int32),
        pltpu.VMEM((_BPW,), jnp.float32),
        pltpu.VMEM((_L,), jnp.float32),
    ],
)


@jax.jit
def kernel(x, target):
    nr = _N // _R
    tgt = target.astype(jnp.int32)
    u = pl.pallas_call(
        _tc_body,
        grid=(nr,),
        in_specs=[
            pl.BlockSpec((1, 1, _R), lambda i: (i, 0, 0)),
            pl.BlockSpec((_R, _SIZE), lambda i: (i, 0)),
        ],
        out_specs=pl.BlockSpec((_R,), lambda i: (i,)),
        out_shape=jax.ShapeDtypeStruct((_N,), jnp.float32),
    )(tgt.reshape(nr, 1, _R), x)
    parts = _sc_finalize(tgt, u)
    return jnp.sum(parts)


# speedup vs baseline: 1.0543x; 1.0167x over previous
import math as _math

import jax
import jax.numpy as jnp
from jax import lax
from jax.experimental import pallas as pl
from jax.experimental.pallas import tpu as pltpu
from jax.experimental.pallas import tpu_sc as plsc

_SIZE = 16384
_N = 4096
_SMOOTH = 0.1
_CONF = 1.0 - _SMOOTH
_S = _SMOOTH / (_SIZE - 2)
_C0 = (_SIZE - 2) * _S * _math.log(_S) + _CONF * _math.log(_CONF)

_R = 256

_info = plsc.get_sparse_core_info()
_NC, _NS, _L = 1, _info.num_subcores, _info.num_lanes
_NW = _NC * _NS
_BPW = _N // _NW


def _tc_body(tgt_ref, x_ref, u_ref):
    t = tgt_ref[0, 0, :]
    x = x_ref[...]
    cols = lax.broadcasted_iota(jnp.int32, (_R, _SIZE), 1)
    td = jnp.where(cols == t[:, None], jnp.float32(_CONF), jnp.float32(_S))
    td = jnp.where(cols == 0, jnp.float32(0.0), td)
    u_ref[...] = jnp.sum(td * x, axis=1)


def _sc_finalize_body(tgt_hbm, u_hbm, out_hbm, tgt_v, u_v, part_v):
    wid = lax.axis_index("s") * _NC + lax.axis_index("c")
    base = wid * _BPW
    pltpu.sync_copy(tgt_hbm.at[pl.ds(base, _BPW)], tgt_v)
    pltpu.sync_copy(u_hbm.at[pl.ds(base, _BPW)], u_v)
    acc = jnp.zeros((_L,), jnp.float32)
    for c in range(_BPW // _L):
        t = tgt_v[pl.ds(c * _L, _L)]
        u = u_v[pl.ds(c * _L, _L)]
        acc = acc + jnp.where(t != 0, jnp.float32(_C0) - u, jnp.float32(0.0))
    part_v[...] = acc
    pltpu.sync_copy(part_v, out_hbm.at[wid])


_sc_finalize = pl.kernel(
    _sc_finalize_body,
    out_type=jax.ShapeDtypeStruct((_NW, _L), jnp.float32),
    mesh=plsc.VectorSubcoreMesh(
        core_axis_name="c", subcore_axis_name="s", num_cores=1),
    scratch_types=[
        pltpu.VMEM((_BPW,), jnp.int32),
        pltpu.VMEM((_BPW,), jnp.float32),
        pltpu.VMEM((_L,), jnp.float32),
    ],
)


@jax.jit
def kernel(x, target):
    nr = _N // _R
    tgt = target.astype(jnp.int32)
    u = pl.pallas_call(
        _tc_body,
        grid=(nr,),
        in_specs=[
            pl.BlockSpec((1, 1, _R), lambda i: (i, 0, 0)),
            pl.BlockSpec((_R, _SIZE), lambda i: (i, 0)),
        ],
        out_specs=pl.BlockSpec((_R,), lambda i: (i,)),
        out_shape=jax.ShapeDtypeStruct((_N,), jnp.float32),
    )(tgt.reshape(nr, 1, _R), x)
    parts = _sc_finalize(tgt, u)
    return jnp.sum(parts)
